# Initial kernel scaffold; baseline (speedup 1.0000x reference)
import jax
import jax.numpy as jnp
from jax.experimental import pallas as pl


def kernel(x):
    raise NotImplementedError("write your pallas kernel here")


# speedup vs baseline: 14.8296x; 14.8296x over previous
import functools

import jax
import jax.numpy as jnp
from jax.experimental import pallas as pl

_ROWS_PER_BLOCK = 8
_BISECT_ITERS = 24


def _sparsemax_block(x_ref, o_ref):
    x = x_ref[...]
    m = jnp.max(x, axis=-1, keepdims=True)
    lo = m - 1.0
    hi = m

    def body(_, carry):
        lo, hi = carry
        mid = 0.5 * (lo + hi)
        s = jnp.sum(jnp.maximum(x - mid, 0.0), axis=-1, keepdims=True)
        pred = s > 1.0
        lo = jnp.where(pred, mid, lo)
        hi = jnp.where(pred, hi, mid)
        return lo, hi

    lo, hi = jax.lax.fori_loop(0, _BISECT_ITERS, body, (lo, hi))
    mid = 0.5 * (lo + hi)
    mask = x > mid
    k = jnp.sum(mask.astype(x.dtype), axis=-1, keepdims=True)
    s = jnp.sum(jnp.where(mask, x, 0.0), axis=-1, keepdims=True)
    tau = (s - 1.0) / k
    o_ref[...] = jnp.maximum(x + tau, 0.0)


@jax.jit
def kernel(x):
    rows, n = x.shape
    grid = rows // _ROWS_PER_BLOCK
    return pl.pallas_call(
        _sparsemax_block,
        grid=(grid,),
        in_specs=[pl.BlockSpec((_ROWS_PER_BLOCK, n), lambda i: (i, 0))],
        out_specs=pl.BlockSpec((_ROWS_PER_BLOCK, n), lambda i: (i, 0)),
        out_shape=jax.ShapeDtypeStruct((rows, n), x.dtype),
    )(x)
